# baseline (speedup 1.0000x reference)
import jax
import jax.numpy as jnp
from jax import lax
from jax.experimental import pallas as pl
from jax.experimental.pallas import tpu as pltpu
from jax.experimental.pallas import tpu_sc as plsc

B, N_ID, N_SYM, D = 4096, 200, 200, 64
PAD_IDX = 0
L = 16
NC, NS = 2, 16
NW = NC * NS
BPW = B // NW
TW = N_ID * D
OW = N_SYM * D
NB = 4
IW = N_SYM + L


def _body(table, pad, idx_hbm, mask_hbm, out,
          ibufs, mbufs, tbufs, obufs, sem_t, sem_w):
    wid = lax.axis_index("s") * NC + lax.axis_index("c")
    tbase = wid * BPW * TW
    obase = wid * BPW * OW
    ibase = wid * BPW * N_SYM

    for par in range(NB):
        pltpu.sync_copy(pad, tbufs[par].at[pl.ds(TW, D)])

    def fetch(b, par):
        pltpu.async_copy(table.at[pl.ds(tbase + b * TW, TW)],
                         tbufs[par].at[pl.ds(0, TW)], sem_t[par])
        pltpu.async_copy(idx_hbm.at[pl.ds(ibase + b * N_SYM, N_SYM)],
                         ibufs[par].at[pl.ds(0, N_SYM)], sem_t[par])
        pltpu.async_copy(mask_hbm.at[pl.ds(ibase + b * N_SYM, N_SYM)],
                         mbufs[par].at[pl.ds(0, N_SYM)], sem_t[par])

    def wait_fetch(par):
        pltpu.make_async_copy(table.at[pl.ds(0, TW)],
                              tbufs[par].at[pl.ds(0, TW)], sem_t[par]).wait()
        pltpu.make_async_copy(idx_hbm.at[pl.ds(0, N_SYM)],
                              ibufs[par].at[pl.ds(0, N_SYM)], sem_t[par]).wait()
        pltpu.make_async_copy(mask_hbm.at[pl.ds(0, N_SYM)],
                              mbufs[par].at[pl.ds(0, N_SYM)], sem_t[par]).wait()

    def put(b, par):
        pltpu.async_copy(obufs[par], out.at[pl.ds(obase + b * OW, OW)],
                         sem_w[par])

    def wait_put(par):
        pltpu.make_async_copy(obufs[par], out.at[pl.ds(0, OW)],
                              sem_w[par]).wait()

    def compute(par):
        ib, mb, tb, ob = ibufs[par], mbufs[par], tbufs[par], obufs[par]

        @plsc.parallel_loop(0, N_SYM, 1, unroll=8)
        def rowloop(s):
            idx_s = ib[pl.ds(s, L)][0]
            m_s = mb[pl.ds(s, L)][0]
            r = jnp.where(m_s != 0, idx_s, jnp.int32(N_ID)) << 6
            o = s << 6
            for k in range(0, D, L):
                ob[pl.ds(o + k, L)] = tb[pl.ds(r + k, L)]

    for j in range(NB - 1):
        fetch(j, j)

    def step(cc, carry):
        for par in range(NB):
            b = cc * NB + par
            nxt = (par + NB - 1) % NB

            @pl.when(b + NB - 1 < BPW)
            def _():
                fetch(b + NB - 1, nxt)

            wait_fetch(par)

            @pl.when(b >= NB)
            def _():
                wait_put(par)

            compute(par)
            put(b, par)
        return carry

    lax.fori_loop(0, BPW // NB, step, 0)
    for par in range(NB):
        wait_put(par)


@jax.jit
def _run(table_flat, pad_row, idx_flat, mask_flat):
    f = pl.kernel(
        _body,
        mesh=plsc.VectorSubcoreMesh(core_axis_name="c", subcore_axis_name="s"),
        out_type=jax.ShapeDtypeStruct((B * N_SYM * D,), jnp.float32),
        scratch_types=[
            [pltpu.VMEM((IW,), jnp.int32) for _ in range(NB)],
            [pltpu.VMEM((IW,), jnp.int32) for _ in range(NB)],
            [pltpu.VMEM((TW + D,), jnp.float32) for _ in range(NB)],
            [pltpu.VMEM((OW,), jnp.float32) for _ in range(NB)],
            [pltpu.SemaphoreType.DMA for _ in range(NB)],
            [pltpu.SemaphoreType.DMA for _ in range(NB)],
        ],
        compiler_params=pltpu.CompilerParams(use_tc_tiling_on_sc=False,
                                             needs_layout_passes=False),
    )
    return f(table_flat, pad_row, idx_flat, mask_flat)


def kernel(encoded_identifiers, identifiers_idxs_of_all_symbols,
           identifiers_idxs_of_all_symbols_mask, special_words_embedding):
    table_flat = encoded_identifiers.reshape(-1)
    pad_row = special_words_embedding[PAD_IDX]
    idx_flat = identifiers_idxs_of_all_symbols.reshape(-1).astype(jnp.int32)
    mask_flat = (identifiers_idxs_of_all_symbols_mask
                 .reshape(-1).astype(jnp.int32))
    out = _run(table_flat, pad_row, idx_flat, mask_flat)
    return out.reshape(B, N_SYM, D)
